# Initial kernel scaffold; baseline (speedup 1.0000x reference)
import jax
import jax.numpy as jnp
from jax.experimental import pallas as pl


def kernel(x, edge_index, gcn_w0, gcn_b0, gcn_w1, gcn_b1, gcn_w2, gcn_b2, skip_w0, skip_b0, skip_w1, skip_b1, skip_w2, skip_b2, pool_w0, pool_b0, pool_w1, pool_b1, cls_w, cls_b):
    raise NotImplementedError("write your pallas kernel here")


# speedup vs baseline: 2.0403x; 2.0403x over previous
import functools

import jax
import jax.numpy as jnp
from jax import lax
from jax.experimental import pallas as pl
from jax.experimental.pallas import tpu as pltpu
from jax.experimental.pallas import tpu_sc as plsc

_N = 2048
_E = 65536
_DF = 128
_H = 32
_P1 = 1024
_P2 = 512
_NCLS = 10


_NC = 2
_NS = 16
_LANES = 16
_ROWS_PP = 512
_PASSES = _N // (_NC * _ROWS_PP)
_EPT = _E // _NS
_CHUNK = 128
_NCHUNK = _EPT // _CHUNK
_ZBUF = 8192
_SP_PER_TILE = _ROWS_PP * _N // _NS


def _adj_body(edge_hbm, a_hbm, src_v, dst_v, idx_v, upd_v, zero_v, spmem):
    c = lax.axis_index("c")
    s = lax.axis_index("s")
    e0 = s * _EPT

    pltpu.sync_copy(edge_hbm.at[0, pl.ds(e0, _EPT)], src_v)
    pltpu.sync_copy(edge_hbm.at[1, pl.ds(e0, _EPT)], dst_v)

    @pl.loop(0, _ZBUF, step=_LANES)
    def _(i):
        zero_v[pl.ds(i, _LANES)] = jnp.zeros((_LANES,), jnp.float32)

    @pl.loop(0, _PASSES)
    def _(p):
        row_base = c * (_PASSES * _ROWS_PP) + p * _ROWS_PP

        @pl.loop(0, _SP_PER_TILE, step=_ZBUF)
        def _(z):
            pltpu.sync_copy(zero_v, spmem.at[pl.ds(s * _SP_PER_TILE + z, _ZBUF)])

        @pl.loop(0, _NCHUNK)
        def _(j):
            @pl.loop(0, _CHUNK, step=_LANES)
            def _(k):
                off = j * _CHUNK + k
                src = src_v[pl.ds(off, _LANES)]
                dst = dst_v[pl.ds(off, _LANES)]
                rel = src - row_base
                inb = (rel >= 0) & (rel < _ROWS_PP)
                row = rel & (_ROWS_PP - 1)
                idx_v[j, pl.ds(k, _LANES)] = row * _N + dst
                upd_v[j, pl.ds(k, _LANES)] = jnp.where(
                    inb, jnp.float32(1.0), jnp.float32(0.0))

        plsc.subcore_barrier()

        @pl.loop(0, _NCHUNK)
        def _(j):
            pltpu.sync_copy(upd_v.at[j], spmem.at[idx_v.at[j]], add=True)

        plsc.subcore_barrier()

        pltpu.sync_copy(
            spmem.at[pl.ds(s * _SP_PER_TILE, _SP_PER_TILE)],
            a_hbm.at[pl.ds(row_base * _N + s * _SP_PER_TILE, _SP_PER_TILE)])

        plsc.subcore_barrier()


def _build_adj(edge_index):
    mesh = plsc.VectorSubcoreMesh(core_axis_name="c", subcore_axis_name="s")
    kern = pl.kernel(
        _adj_body,
        out_type=jax.ShapeDtypeStruct((_N * _N,), jnp.float32),
        mesh=mesh,
        scratch_types=[
            pltpu.VMEM((_EPT,), jnp.int32),
            pltpu.VMEM((_EPT,), jnp.int32),
            pltpu.VMEM((_NCHUNK, _CHUNK), jnp.int32),
            pltpu.VMEM((_NCHUNK, _CHUNK), jnp.float32),
            pltpu.VMEM((_ZBUF,), jnp.float32),
            pltpu.VMEM_SHARED((_ROWS_PP * _N,), jnp.float32),
        ],
    )
    return kern(edge_index)


_RB = 256
_G1 = _N // _RB


def _prep_body(a_ref, x_ref, xd_ref, dinv_ref, sa2_ref):
    i = pl.program_id(0)

    @pl.when(i == 0)
    def _():
        sa2_ref[...] = jnp.zeros_like(sa2_ref)

    a = a_ref[...]
    deg = jnp.sum(a, axis=1, keepdims=True) + 1.0
    dinv = lax.rsqrt(deg)
    dinv_ref[...] = dinv
    xd_ref[...] = dinv * x_ref[...]
    sa2_ref[...] += jnp.sum(a * a)


def _prep(a, x):
    return pl.pallas_call(
        _prep_body,
        grid=(_G1,),
        in_specs=[
            pl.BlockSpec((_RB, _N), lambda i: (i, 0)),
            pl.BlockSpec((_RB, _DF), lambda i: (i, 0)),
        ],
        out_specs=[
            pl.BlockSpec((_RB, _DF), lambda i: (i, 0)),
            pl.BlockSpec((_RB, 1), lambda i: (i, 0)),
            pl.BlockSpec((1, 1), lambda i: (0, 0)),
        ],
        out_shape=[
            jax.ShapeDtypeStruct((_N, _DF), jnp.float32),
            jax.ShapeDtypeStruct((_N, 1), jnp.float32),
            jax.ShapeDtypeStruct((1, 1), jnp.float32),
        ],
    )(a, x)


def _s1a_body(a_ref, xd_ref, x_ref, xdb_ref, dinv_ref,
              pw_ref, pb_ref, gw_ref, gb_ref, sw_ref, sb_ref,
              s_ref, x1_ref, outx_ref, gram_ref, ent_ref):
    i = pl.program_id(0)

    @pl.when(i == 0)
    def _():
        outx_ref[...] = jnp.zeros_like(outx_ref)
        gram_ref[...] = jnp.zeros_like(gram_ref)
        ent_ref[...] = jnp.zeros_like(ent_ref)

    dinv = dinv_ref[...]
    t = jnp.dot(a_ref[...], xd_ref[...], preferred_element_type=jnp.float32)
    t = t + xdb_ref[...]

    s_pre = dinv * jnp.dot(t, pw_ref[...], preferred_element_type=jnp.float32)
    s_pre = s_pre + pb_ref[...]
    m = jnp.max(s_pre, axis=-1, keepdims=True)
    e = jnp.exp(s_pre - m)
    s_soft = e / jnp.sum(e, axis=-1, keepdims=True)
    s_ref[...] = s_soft

    xg = dinv * jnp.dot(t, gw_ref[...], preferred_element_type=jnp.float32)
    xg = xg + gb_ref[...]
    x1 = jax.nn.relu(
        xg + jnp.dot(x_ref[...], sw_ref[...], preferred_element_type=jnp.float32)
        + sb_ref[...])
    x1_ref[...] = x1

    dn = (((0,), (0,)), ((), ()))
    outx_ref[...] += lax.dot_general(
        s_soft, x1, dn, preferred_element_type=jnp.float32)
    gram_ref[...] += lax.dot_general(
        s_soft, s_soft, dn, preferred_element_type=jnp.float32)
    ent_ref[...] += jnp.sum(-s_soft * jnp.log(s_soft + 1e-15))


def _stage1a(a, xd, x, dinv, pw, pb, gw, gb, sw, sb):
    return pl.pallas_call(
        _s1a_body,
        grid=(_G1,),
        in_specs=[
            pl.BlockSpec((_RB, _N), lambda i: (i, 0)),
            pl.BlockSpec((_N, _DF), lambda i: (0, 0)),
            pl.BlockSpec((_RB, _DF), lambda i: (i, 0)),
            pl.BlockSpec((_RB, _DF), lambda i: (i, 0)),
            pl.BlockSpec((_RB, 1), lambda i: (i, 0)),
            pl.BlockSpec((_DF, _P1), lambda i: (0, 0)),
            pl.BlockSpec((1, _P1), lambda i: (0, 0)),
            pl.BlockSpec((_DF, _H), lambda i: (0, 0)),
            pl.BlockSpec((1, _H), lambda i: (0, 0)),
            pl.BlockSpec((_DF, _H), lambda i: (0, 0)),
            pl.BlockSpec((1, _H), lambda i: (0, 0)),
        ],
        out_specs=[
            pl.BlockSpec((_RB, _P1), lambda i: (i, 0)),
            pl.BlockSpec((_RB, _H), lambda i: (i, 0)),
            pl.BlockSpec((_P1, _H), lambda i: (0, 0)),
            pl.BlockSpec((_P1, _P1), lambda i: (0, 0)),
            pl.BlockSpec((1, 1), lambda i: (0, 0)),
        ],
        out_shape=[
            jax.ShapeDtypeStruct((_N, _P1), jnp.float32),
            jax.ShapeDtypeStruct((_N, _H), jnp.float32),
            jax.ShapeDtypeStruct((_P1, _H), jnp.float32),
            jax.ShapeDtypeStruct((_P1, _P1), jnp.float32),
            jax.ShapeDtypeStruct((1, 1), jnp.float32),
        ],
    )(a, xd, x, xd, dinv, pw, pb, gw, gb, sw, sb)


def _s1b_body(a_ref, sfull_ref, sblk_ref, adj_ref, cross_ref):
    i = pl.program_id(0)

    @pl.when(i == 0)
    def _():
        adj_ref[...] = jnp.zeros_like(adj_ref)
        cross_ref[...] = jnp.zeros_like(cross_ref)

    b = jnp.dot(a_ref[...], sfull_ref[...], preferred_element_type=jnp.float32)
    sblk = sblk_ref[...]
    dn = (((0,), (0,)), ((), ()))
    adj_ref[...] += lax.dot_general(
        sblk, b, dn, preferred_element_type=jnp.float32)
    cross_ref[...] += jnp.sum(sblk * b)


def _stage1b(a, s_soft):
    return pl.pallas_call(
        _s1b_body,
        grid=(_G1,),
        in_specs=[
            pl.BlockSpec((_RB, _N), lambda i: (i, 0)),
            pl.BlockSpec((_N, _P1), lambda i: (0, 0)),
            pl.BlockSpec((_RB, _P1), lambda i: (i, 0)),
        ],
        out_specs=[
            pl.BlockSpec((_P1, _P1), lambda i: (0, 0)),
            pl.BlockSpec((1, 1), lambda i: (0, 0)),
        ],
        out_shape=[
            jax.ShapeDtypeStruct((_P1, _P1), jnp.float32),
            jax.ShapeDtypeStruct((1, 1), jnp.float32),
        ],
    )(a, s_soft, s_soft)


def _s2_body(x2_ref, a2_ref, gram_ref, ent1_ref, sa2_ref, cross_ref,
             pw_ref, pb_ref, gw1_ref, gb1_ref, sw1_ref, sb1_ref,
             gw2_ref, gb2_ref, sw2_ref, sb2_ref, cw_ref, cb_ref,
             out_ref, l1_ref, l2_ref):
    f32 = jnp.float32
    dn = (((0,), (0,)), ((), ()))

    gram = gram_ref[...]
    num1 = sa2_ref[0, 0] - 2.0 * cross_ref[0, 0] + jnp.sum(gram * gram)
    l1a = jnp.sqrt(jnp.maximum(num1, 0.0)) / (jnp.float32(_N) * jnp.float32(_N))
    l2a = ent1_ref[0, 0] / jnp.float32(_N)

    x2 = x2_ref[...]
    a2 = a2_ref[...]

    deg2 = jnp.sum(a2, axis=1, keepdims=True) + 1.0
    dinv2 = lax.rsqrt(deg2)
    xd2 = dinv2 * x2
    t2 = jnp.dot(a2, xd2, preferred_element_type=f32) + xd2

    s2p = dinv2 * jnp.dot(t2, pw_ref[...], preferred_element_type=f32)
    s2p = s2p + pb_ref[...]
    m = jnp.max(s2p, axis=-1, keepdims=True)
    e = jnp.exp(s2p - m)
    s2 = e / jnp.sum(e, axis=-1, keepdims=True)

    xg2 = dinv2 * jnp.dot(t2, gw1_ref[...], preferred_element_type=f32)
    x2b = jax.nn.relu(
        xg2 + gb1_ref[...]
        + jnp.dot(x2, sw1_ref[...], preferred_element_type=f32) + sb1_ref[...])

    b2 = jnp.dot(a2, s2, preferred_element_type=f32)
    x3 = lax.dot_general(s2, x2b, dn, preferred_element_type=f32)
    a3 = lax.dot_general(s2, b2, dn, preferred_element_type=f32)

    gram2 = lax.dot_general(s2, s2, dn, preferred_element_type=f32)
    num2 = (jnp.sum(a2 * a2) - 2.0 * jnp.sum(s2 * b2)
            + jnp.sum(gram2 * gram2))
    l1b = jnp.sqrt(jnp.maximum(num2, 0.0)) / (jnp.float32(_P1) * jnp.float32(_P1))
    l2b = jnp.sum(-s2 * jnp.log(s2 + 1e-15)) / jnp.float32(_P1)

    deg3 = jnp.sum(a3, axis=1, keepdims=True) + 1.0
    dinv3 = lax.rsqrt(deg3)
    xd3 = dinv3 * x3
    t3 = jnp.dot(a3, xd3, preferred_element_type=f32) + xd3
    xg3 = dinv3 * jnp.dot(t3, gw2_ref[...], preferred_element_type=f32)
    x4 = jax.nn.relu(
        xg3 + gb2_ref[...]
        + jnp.dot(x3, sw2_ref[...], preferred_element_type=f32) + sb2_ref[...])

    pooled = jnp.sum(x4, axis=0, keepdims=True) / jnp.float32(_P2)
    out_ref[...] = (jnp.dot(pooled, cw_ref[...], preferred_element_type=f32)
                    + cb_ref[...])
    l1_ref[...] = jnp.full((1, 1), 0.0, f32) + (l1a + l1b)
    l2_ref[...] = jnp.full((1, 1), 0.0, f32) + (l2a + l2b)


def _stage2(x2, a2, gram, ent1, sa2, cross,
            pw1, pb1, gw1, gb1, sw1, sb1, gw2, gb2, sw2, sb2, cw, cb):
    return pl.pallas_call(
        _s2_body,
        out_shape=[
            jax.ShapeDtypeStruct((1, _NCLS), jnp.float32),
            jax.ShapeDtypeStruct((1, 1), jnp.float32),
            jax.ShapeDtypeStruct((1, 1), jnp.float32),
        ],
    )(x2, a2, gram, ent1, sa2, cross,
      pw1, pb1, gw1, gb1, sw1, sb1, gw2, gb2, sw2, sb2, cw, cb)


def kernel(x, edge_index, gcn_w0, gcn_b0, gcn_w1, gcn_b1, gcn_w2, gcn_b2,
           skip_w0, skip_b0, skip_w1, skip_b1, skip_w2, skip_b2,
           pool_w0, pool_b0, pool_w1, pool_b1, cls_w, cls_b):
    a = _build_adj(edge_index).reshape(_N, _N)
    xd, dinv, sa2 = _prep(a, x)
    s_soft, x1, out_x, gram, ent1 = _stage1a(
        a, xd, x, dinv,
        pool_w0, pool_b0.reshape(1, _P1),
        gcn_w0, gcn_b0.reshape(1, _H),
        skip_w0, skip_b0.reshape(1, _H))
    del x1
    out_adj, cross = _stage1b(a, s_soft)
    out, l1, l2 = _stage2(
        out_x, out_adj, gram, ent1, sa2, cross,
        pool_w1, pool_b1.reshape(1, _P2),
        gcn_w1, gcn_b1.reshape(1, _H),
        skip_w1, skip_b1.reshape(1, _H),
        gcn_w2, gcn_b2.reshape(1, _H),
        skip_w2, skip_b2.reshape(1, _H),
        cls_w, cls_b.reshape(1, _NCLS))
    return out, l1[0, 0], l2[0, 0]
